# baseline (speedup 1.0000x reference)
import functools

import jax
import jax.numpy as jnp
import numpy as np
from jax import lax
from jax.experimental import pallas as pl
from jax.experimental.pallas import tpu as pltpu
from jax.experimental.pallas import tpu_sc as plsc

N = 10000
E = 320000
MUL = 16
H = 64
WN = 256

NPAD = 10240
ROWS_PER_TILE = NPAD // 16
BLK = 128
NBLK = E // BLK
NW = 32
BLK_PER_W = NBLK // NW
REM = NBLK - NW * BLK_PER_W
GK = 6
NG = BLK_PER_W // GK
NPAIR = (NG - 1) // 2
EPW = BLK_PER_W * BLK


@functools.lru_cache(maxsize=1)
def _sc_kernels():
    mesh = plsc.VectorSubcoreMesh(core_axis_name="c", subcore_axis_name="s")
    sc_params = pltpu.CompilerParams(use_tc_tiling_on_sc=False)

    def sc_gather(table_hbm, eidx_hbm, out_hbm, idx_flat, rows_v,
                  sem_a, sem_b):
        cid = lax.axis_index("c")
        sid = lax.axis_index("s")
        wid = sid * 2 + cid
        base = wid * BLK_PER_W

        pltpu.sync_copy(eidx_hbm.at[0, pl.ds(base * BLK, EPW)],
                        idx_flat.at[pl.ds(0, EPW)])

        @pl.when(wid < REM)
        def _():
            pltpu.sync_copy(
                eidx_hbm.at[0, pl.ds((NW * BLK_PER_W + wid) * BLK, BLK)],
                idx_flat.at[pl.ds(EPW, BLK)])

        def fire(g, slot, sem):
            for k in range(GK):
                pltpu.async_copy(
                    table_hbm.at[idx_flat.at[pl.ds((g * GK + k) * BLK, BLK)]],
                    rows_v.at[slot, k], sem)

        def drain(slot, sem):
            for k in range(GK):
                pltpu.make_async_copy(table_hbm.at[idx_flat.at[pl.ds(0, BLK)]],
                                      rows_v.at[slot, k], sem).wait()

        def writeout(g, slot):
            pltpu.sync_copy(rows_v.at[slot],
                            out_hbm.at[pl.ds(base + g * GK, GK)])

        fire(0, 0, sem_a)

        def pair(p, carry):
            g0 = 2 * p
            fire(g0 + 1, 1, sem_b)
            drain(0, sem_a)
            writeout(g0, 0)
            fire(g0 + 2, 0, sem_a)
            drain(1, sem_b)
            writeout(g0 + 1, 1)
            return carry

        lax.fori_loop(0, NPAIR, pair, 0)
        drain(0, sem_a)
        writeout(NG - 1, 0)

        @pl.when(wid < REM)
        def _():
            blk = NW * BLK_PER_W + wid
            pltpu.async_copy(table_hbm.at[idx_flat.at[pl.ds(EPW, BLK)]],
                             rows_v.at[0, 0], sem_a).wait()
            pltpu.sync_copy(rows_v.at[0, 0], out_hbm.at[blk])

    sc_gather = pl.kernel(
        sc_gather,
        out_type=jax.ShapeDtypeStruct((NBLK, BLK, MUL), jnp.float32),
        mesh=mesh,
        compiler_params=sc_params,
        scratch_types=[
            pltpu.VMEM((EPW + BLK,), jnp.int32),
            pltpu.VMEM((2, GK, BLK, MUL), jnp.float32),
            pltpu.SemaphoreType.DMA,
            pltpu.SemaphoreType.DMA,
        ],
    )

    def sc_scatter(tp_hbm, eidx_hbm, sums_out, cnts_out,
                   idx_all, rows_v, ones_v, zbuf,
                   sums_sh, cnts_sh, sem_a, sem_b, sem_s):
        cid = lax.axis_index("c")
        sid = lax.axis_index("s")
        wid = sid * 2 + cid
        base = wid * BLK_PER_W

        def idx_fire(b, carry):
            pltpu.async_copy(eidx_hbm.at[1, pl.ds((base + b) * BLK, BLK)],
                             idx_all.at[b], sem_s)
            return carry

        lax.fori_loop(0, BLK_PER_W, idx_fire, 0)

        @pl.when(wid < REM)
        def _():
            pltpu.async_copy(
                eidx_hbm.at[1, pl.ds((NW * BLK_PER_W + wid) * BLK, BLK)],
                idx_all.at[BLK_PER_W], sem_s).wait()

        def fill(i, carry):
            ones_v[i] = jnp.ones((MUL,), jnp.float32)
            return carry

        lax.fori_loop(0, BLK, fill, 0)

        def zero(i, carry):
            zbuf[i] = jnp.zeros((MUL,), jnp.float32)
            return carry

        lax.fori_loop(0, ROWS_PER_TILE, zero, 0)

        def idx_drain(b, carry):
            pltpu.make_async_copy(eidx_hbm.at[1, pl.ds(0, BLK)],
                                  idx_all.at[0], sem_s).wait()
            return carry

        lax.fori_loop(0, BLK_PER_W, idx_drain, 0)

        row0 = sid * ROWS_PER_TILE
        pltpu.sync_copy(zbuf, sums_sh.at[pl.ds(row0, ROWS_PER_TILE)])
        pltpu.sync_copy(zbuf, cnts_sh.at[pl.ds(row0, ROWS_PER_TILE)])
        plsc.subcore_barrier()

        def load(g, slot, sem):
            pltpu.async_copy(tp_hbm.at[pl.ds(base + g * GK, GK)],
                             rows_v.at[slot], sem)

        def loadwait(slot, sem):
            pltpu.make_async_copy(tp_hbm.at[pl.ds(0, GK)],
                                  rows_v.at[slot], sem).wait()

        def streams(g, slot):
            descs = []
            for k in range(GK):
                idx_row = idx_all.at[g * GK + k]
                descs.append(pltpu.async_copy(
                    rows_v.at[slot, k], sums_sh.at[idx_row], sem_s, add=True))
                descs.append(pltpu.async_copy(
                    ones_v, cnts_sh.at[idx_row], sem_s, add=True))
            for d in descs:
                d.wait()

        load(0, 0, sem_a)

        def pair(p, carry):
            g0 = 2 * p
            load(g0 + 1, 1, sem_b)
            loadwait(0, sem_a)
            streams(g0, 0)
            load(g0 + 2, 0, sem_a)
            loadwait(1, sem_b)
            streams(g0 + 1, 1)
            return carry

        lax.fori_loop(0, NPAIR, pair, 0)
        loadwait(0, sem_a)
        streams(NG - 1, 0)

        @pl.when(wid < REM)
        def _():
            blk = NW * BLK_PER_W + wid
            pltpu.sync_copy(tp_hbm.at[blk], rows_v.at[0, 0])
            idx_row = idx_all.at[BLK_PER_W]
            d1 = pltpu.async_copy(rows_v.at[0, 0], sums_sh.at[idx_row],
                                  sem_s, add=True)
            d2 = pltpu.async_copy(ones_v, cnts_sh.at[idx_row], sem_s, add=True)
            d1.wait()
            d2.wait()

        plsc.subcore_barrier()
        pltpu.sync_copy(sums_sh.at[pl.ds(row0, ROWS_PER_TILE)],
                        sums_out.at[cid, pl.ds(row0, ROWS_PER_TILE)])
        pltpu.sync_copy(cnts_sh.at[pl.ds(row0, ROWS_PER_TILE)],
                        cnts_out.at[cid, pl.ds(row0, ROWS_PER_TILE)])

    sc_scatter = pl.kernel(
        sc_scatter,
        out_type=(
            jax.ShapeDtypeStruct((2, NPAD, MUL), jnp.float32),
            jax.ShapeDtypeStruct((2, NPAD, MUL), jnp.float32),
        ),
        mesh=mesh,
        compiler_params=sc_params,
        scratch_types=[
            pltpu.VMEM((BLK_PER_W + 2, BLK), jnp.int32),
            pltpu.VMEM((2, GK, BLK, MUL), jnp.float32),
            pltpu.VMEM((BLK, MUL), jnp.float32),
            pltpu.VMEM((ROWS_PER_TILE, MUL), jnp.float32),
            pltpu.VMEM_SHARED((NPAD, MUL), jnp.float32),
            pltpu.VMEM_SHARED((NPAD, MUL), jnp.float32),
            pltpu.SemaphoreType.DMA,
            pltpu.SemaphoreType.DMA,
            pltpu.SemaphoreType.DMA,
        ],
    )

    return sc_gather, sc_scatter


T_EDGE = 2560
TP = T_EDGE // 8
TS = T_EDGE // 128


def _tc_edge_body(ef_ref, x_ref, sh_ref, w1_ref, b1_ref, w2_ref, b2_ref,
                  s_ref, m_ref, out_ref):
    h = lax.dot_general(ef_ref[...].astype(jnp.bfloat16), w1_ref[...],
                        (((0,), (0,)), ((), ())),
                        preferred_element_type=jnp.float32)
    h16 = jnp.maximum(h + b1_ref[...], 0.0).astype(jnp.bfloat16)
    h_s = jnp.transpose(h16.reshape(TP, 8, H), (1, 0, 2)).reshape(T_EDGE, H)
    w = (jnp.dot(h_s, w2_ref[...], preferred_element_type=jnp.float32)
         .astype(jnp.bfloat16) + b2_ref[...])
    xp = x_ref[...].astype(jnp.bfloat16)
    x16 = jnp.concatenate(
        [xp[:, MUL * g:MUL * (g + 1)] for g in range(8)], axis=0)
    xrep = jnp.concatenate([x16] * MUL, axis=1)
    tp = jnp.dot(xrep * w, s_ref[...], preferred_element_type=jnp.float32)
    packed = jnp.concatenate(
        [tp[TP * g:TP * (g + 1), :] for g in range(8)], axis=1)
    shp = sh_ref[0]
    shb = jnp.concatenate(
        [jnp.dot(shp, m_ref[:, 128 * u:128 * (u + 1)],
                 preferred_element_type=jnp.float32).reshape(TS, 1, 128)
         for u in range(16)], axis=1).reshape(TP, 128)
    out_ref[...] = packed * (shb * (1.0 / np.sqrt(MUL)))


_tc_edge = pl.pallas_call(
    _tc_edge_body,
    grid=(E // T_EDGE,),
    compiler_params=pltpu.CompilerParams(fuse_transposed_lhs_in_matmul=True),
    in_specs=[
        pl.BlockSpec((H, T_EDGE), lambda i: (0, i)),
        pl.BlockSpec((TP, 128), lambda i: (i, 0)),
        pl.BlockSpec((1, TS, 128), lambda i: (i, 0, 0)),
        pl.BlockSpec((H, H), lambda i: (0, 0)),
        pl.BlockSpec((1, H), lambda i: (0, 0)),
        pl.BlockSpec((H, WN), lambda i: (0, 0)),
        pl.BlockSpec((1, WN), lambda i: (0, 0)),
        pl.BlockSpec((WN, MUL), lambda i: (0, 0)),
        pl.BlockSpec((128, 16 * 128), lambda i: (0, 0)),
    ],
    out_specs=pl.BlockSpec((TP, 128), lambda i: (i, 0)),
    out_shape=jax.ShapeDtypeStruct((E // 8, 128), jnp.float32),
)

_S_np = np.zeros((MUL * MUL, MUL), np.float32)
for _i in range(MUL):
    for _j in range(MUL):
        _S_np[_j * MUL + _i, _j] = 1.0
_PERM_np = np.array([(_ij % MUL) * MUL + _ij // MUL
                     for _ij in range(MUL * MUL)], np.int32)
_M_np = np.zeros((128, 16 * 128), np.float32)
for _u in range(16):
    for _g in range(8):
        for _c in range(MUL):
            _M_np[8 * _u + _g, _u * 128 + 16 * _g + _c] = 1.0
_G_np = np.zeros((128, MUL), np.float32)
for _g in range(128 // MUL):
    for _c in range(MUL):
        _G_np[_g * MUL + _c, _c] = 1.0

NR = NPAD * MUL // 128


def _tc_fin_body(ps_ref, pc_ref, atom_ref, bnw_ref, bnb_ref, g_ref, gt_ref,
                 out_ref):
    s = ps_ref[0] + ps_ref[1]
    cnt = pc_ref[0] + pc_ref[1]
    pre = s / jnp.maximum(cnt, 1.0) + atom_ref[...]
    colsum = jnp.sum(pre, axis=0, keepdims=True)
    colsq = jnp.sum(pre * pre, axis=0, keepdims=True)
    chs = jnp.dot(colsum, g_ref[...], preferred_element_type=jnp.float32)
    chsq = jnp.dot(colsq, g_ref[...], preferred_element_type=jnp.float32)
    mean = chs / float(N)
    var = chsq / float(N) - mean * mean
    inv = lax.rsqrt(var + 1e-5)
    scale = bnw_ref[...] * inv
    shift = bnb_ref[...] - mean * scale
    out_ref[...] = (pre * jnp.dot(scale, gt_ref[...], preferred_element_type=jnp.float32)
                    + jnp.dot(shift, gt_ref[...], preferred_element_type=jnp.float32))


_tc_fin = pl.pallas_call(
    _tc_fin_body,
    in_specs=[
        pl.BlockSpec((2, NR, 128), lambda: (0, 0, 0)),
        pl.BlockSpec((2, NR, 128), lambda: (0, 0, 0)),
        pl.BlockSpec((NR, 128), lambda: (0, 0)),
        pl.BlockSpec((1, MUL), lambda: (0, 0)),
        pl.BlockSpec((1, MUL), lambda: (0, 0)),
        pl.BlockSpec((128, MUL), lambda: (0, 0)),
        pl.BlockSpec((MUL, 128), lambda: (0, 0)),
    ],
    out_specs=pl.BlockSpec((NR, 128), lambda: (0, 0)),
    out_shape=jax.ShapeDtypeStruct((NR, 128), jnp.float32),
)


def kernel(atom_features, edge_features, edge_sh, edge_index, W1, b1, W2, b2,
           bn_weight, bn_bias):
    sc_gather, sc_scatter = _sc_kernels()
    x3 = sc_gather(atom_features, edge_index)
    perm = jnp.asarray(_PERM_np)
    tpp = _tc_edge(edge_features.T,
                   x3.reshape(E // 8, 128),
                   edge_sh[:, 0].reshape(E // T_EDGE, TS, 128),
                   W1.astype(jnp.bfloat16), b1.reshape(1, H),
                   W2[:, perm].astype(jnp.bfloat16),
                   b2[perm].reshape(1, WN).astype(jnp.bfloat16),
                   jnp.asarray(_S_np, jnp.bfloat16),
                   jnp.asarray(_M_np))
    psums, pcnts = sc_scatter(tpp.reshape(NBLK, BLK, MUL), edge_index)

    atom_pad = jnp.pad(atom_features, ((0, NPAD - N), (0, 0))).reshape(NR, 128)
    out = _tc_fin(psums.reshape(2, NR, 128), pcnts.reshape(2, NR, 128),
                  atom_pad, bn_weight.reshape(1, MUL), bn_bias.reshape(1, MUL),
                  jnp.asarray(_G_np), jnp.asarray(_G_np.T))
    return out.reshape(NPAD, MUL)[:N]


# speedup vs baseline: 1.0212x; 1.0212x over previous
import functools

import jax
import jax.numpy as jnp
import numpy as np
from jax import lax
from jax.experimental import pallas as pl
from jax.experimental.pallas import tpu as pltpu
from jax.experimental.pallas import tpu_sc as plsc

N = 10000
E = 320000
MUL = 16
H = 64
WN = 256

NPAD = 10240
ROWS_PER_TILE = NPAD // 16
BLK = 128
NBLK = E // BLK
NW = 32

T_EDGE = 2560
TP = T_EDGE // 8
TS = T_EDGE // 128

TILES_A = 64
TILES_B = (E // T_EDGE) - TILES_A
NBLK_A = TILES_A * T_EDGE // BLK
NBLK_B = NBLK - NBLK_A
GKMAX = 5


def _half_plan(nblk):
    nblk_w = nblk // NW
    rem = nblk - nblk_w * NW
    groups = [GKMAX] * (nblk_w // GKMAX)
    if nblk_w % GKMAX:
        groups.append(nblk_w % GKMAX)
    return nblk_w, rem, groups


@functools.lru_cache(maxsize=1)
def _sc_kernels():
    mesh = plsc.VectorSubcoreMesh(core_axis_name="c", subcore_axis_name="s")
    sc_params = pltpu.CompilerParams(use_tc_tiling_on_sc=False)

    def make_gather(base_blk, nblk):
        nblk_w, rem, groups = _half_plan(nblk)
        epw = nblk_w * BLK
        gs = np.cumsum([0] + groups).tolist()
        ng = len(groups)

        def body(table_hbm, eidx_hbm, out_hbm, idx_flat, rows_v, sem_a, sem_b):
            cid = lax.axis_index("c")
            sid = lax.axis_index("s")
            wid = sid * 2 + cid
            lbase = wid * nblk_w
            gbase = base_blk + lbase

            pltpu.sync_copy(eidx_hbm.at[0, pl.ds(gbase * BLK, epw)],
                            idx_flat.at[pl.ds(0, epw)])

            @pl.when(wid < rem)
            def _():
                blk_x = base_blk + NW * nblk_w + wid
                pltpu.sync_copy(eidx_hbm.at[0, pl.ds(blk_x * BLK, BLK)],
                                idx_flat.at[pl.ds(epw, BLK)])

            sems = (sem_a, sem_b)

            def fire(gi, slot):
                for k in range(groups[gi]):
                    pltpu.async_copy(
                        table_hbm.at[
                            idx_flat.at[pl.ds((gs[gi] + k) * BLK, BLK)]],
                        rows_v.at[slot, k], sems[slot])

            def drain(gi, slot):
                for k in range(groups[gi]):
                    pltpu.make_async_copy(
                        table_hbm.at[idx_flat.at[pl.ds(0, BLK)]],
                        rows_v.at[slot, k], sems[slot]).wait()

            def writeout(gi, slot):
                gk = groups[gi]
                pltpu.sync_copy(rows_v.at[slot, pl.ds(0, gk)],
                                out_hbm.at[pl.ds(lbase + gs[gi], gk)])

            fire(0, 0)
            for g in range(ng):
                slot = g & 1
                if g + 1 < ng:
                    fire(g + 1, 1 - slot)
                drain(g, slot)
                writeout(g, slot)

            @pl.when(wid < rem)
            def _():
                pltpu.async_copy(table_hbm.at[idx_flat.at[pl.ds(epw, BLK)]],
                                 rows_v.at[0, 0], sem_a).wait()
                pltpu.sync_copy(rows_v.at[0, 0],
                                out_hbm.at[NW * nblk_w + wid])

        return pl.kernel(
            body,
            out_type=jax.ShapeDtypeStruct((nblk, BLK, MUL), jnp.float32),
            mesh=mesh,
            compiler_params=sc_params,
            scratch_types=[
                pltpu.VMEM((epw + BLK,), jnp.int32),
                pltpu.VMEM((2, GKMAX, BLK, MUL), jnp.float32),
                pltpu.SemaphoreType.DMA,
                pltpu.SemaphoreType.DMA,
            ],
        )

    def make_scatter(base_blk, nblk):
        nblk_w, rem, groups = _half_plan(nblk)
        gs = np.cumsum([0] + groups).tolist()
        ng = len(groups)

        def body(tp_hbm, eidx_hbm, sums_out, cnts_out,
                 idx_all, rows_v, ones_v, zbuf,
                 sums_sh, cnts_sh, sem_a, sem_b, sem_s):
            cid = lax.axis_index("c")
            sid = lax.axis_index("s")
            wid = sid * 2 + cid
            lbase = wid * nblk_w
            gbase = base_blk + lbase

            def idx_fire(b, carry):
                pltpu.async_copy(eidx_hbm.at[1, pl.ds((gbase + b) * BLK, BLK)],
                                 idx_all.at[b], sem_s)
                return carry

            lax.fori_loop(0, nblk_w, idx_fire, 0)

            @pl.when(wid < rem)
            def _():
                blk_x = base_blk + NW * nblk_w + wid
                pltpu.async_copy(eidx_hbm.at[1, pl.ds(blk_x * BLK, BLK)],
                                 idx_all.at[nblk_w], sem_s).wait()

            def fill(i, carry):
                ones_v[i] = jnp.ones((MUL,), jnp.float32)
                return carry

            lax.fori_loop(0, BLK, fill, 0)

            def zero(i, carry):
                zbuf[i] = jnp.zeros((MUL,), jnp.float32)
                return carry

            lax.fori_loop(0, ROWS_PER_TILE, zero, 0)

            def idx_drain(b, carry):
                pltpu.make_async_copy(eidx_hbm.at[1, pl.ds(0, BLK)],
                                      idx_all.at[0], sem_s).wait()
                return carry

            lax.fori_loop(0, nblk_w, idx_drain, 0)

            row0 = sid * ROWS_PER_TILE
            pltpu.sync_copy(zbuf, sums_sh.at[pl.ds(row0, ROWS_PER_TILE)])
            pltpu.sync_copy(zbuf, cnts_sh.at[pl.ds(row0, ROWS_PER_TILE)])
            plsc.subcore_barrier()

            sems = (sem_a, sem_b)

            def load(gi, slot):
                gk = groups[gi]
                pltpu.async_copy(tp_hbm.at[pl.ds(lbase + gs[gi], gk)],
                                 rows_v.at[slot, pl.ds(0, gk)], sems[slot])

            def loadwait(gi, slot):
                gk = groups[gi]
                pltpu.make_async_copy(tp_hbm.at[pl.ds(0, gk)],
                                      rows_v.at[slot, pl.ds(0, gk)],
                                      sems[slot]).wait()

            def streams(gi, slot):
                descs = []
                for k in range(groups[gi]):
                    idx_row = idx_all.at[gs[gi] + k]
                    descs.append(pltpu.async_copy(
                        rows_v.at[slot, k], sums_sh.at[idx_row],
                        sem_s, add=True))
                    descs.append(pltpu.async_copy(
                        ones_v, cnts_sh.at[idx_row], sem_s, add=True))
                for d in descs:
                    d.wait()

            load(0, 0)
            for g in range(ng):
                slot = g & 1
                if g + 1 < ng:
                    load(g + 1, 1 - slot)
                loadwait(g, slot)
                streams(g, slot)

            @pl.when(wid < rem)
            def _():
                pltpu.sync_copy(tp_hbm.at[NW * nblk_w + wid], rows_v.at[0, 0])
                idx_row = idx_all.at[nblk_w]
                d1 = pltpu.async_copy(rows_v.at[0, 0], sums_sh.at[idx_row],
                                      sem_s, add=True)
                d2 = pltpu.async_copy(ones_v, cnts_sh.at[idx_row],
                                      sem_s, add=True)
                d1.wait()
                d2.wait()

            plsc.subcore_barrier()
            pltpu.sync_copy(sums_sh.at[pl.ds(row0, ROWS_PER_TILE)],
                            sums_out.at[cid, pl.ds(row0, ROWS_PER_TILE)])
            pltpu.sync_copy(cnts_sh.at[pl.ds(row0, ROWS_PER_TILE)],
                            cnts_out.at[cid, pl.ds(row0, ROWS_PER_TILE)])

        return pl.kernel(
            body,
            out_type=(
                jax.ShapeDtypeStruct((2, NPAD, MUL), jnp.float32),
                jax.ShapeDtypeStruct((2, NPAD, MUL), jnp.float32),
            ),
            mesh=mesh,
            compiler_params=sc_params,
            scratch_types=[
                pltpu.VMEM((nblk_w + 2, BLK), jnp.int32),
                pltpu.VMEM((2, GKMAX, BLK, MUL), jnp.float32),
                pltpu.VMEM((BLK, MUL), jnp.float32),
                pltpu.VMEM((ROWS_PER_TILE, MUL), jnp.float32),
                pltpu.VMEM_SHARED((NPAD, MUL), jnp.float32),
                pltpu.VMEM_SHARED((NPAD, MUL), jnp.float32),
                pltpu.SemaphoreType.DMA,
                pltpu.SemaphoreType.DMA,
                pltpu.SemaphoreType.DMA,
            ],
        )

    return (make_gather(0, NBLK_A), make_gather(NBLK_A, NBLK_B),
            make_scatter(0, NBLK_A), make_scatter(NBLK_A, NBLK_B))


def _tc_edge_body(ef_ref, x_ref, sh_ref, w1_ref, b1_ref, w2_ref, b2_ref,
                  s_ref, m_ref, out_ref):
    h = lax.dot_general(ef_ref[...].astype(jnp.bfloat16), w1_ref[...],
                        (((0,), (0,)), ((), ())),
                        preferred_element_type=jnp.float32)
    h16 = jnp.maximum(h + b1_ref[...], 0.0).astype(jnp.bfloat16)
    h_s = jnp.transpose(h16.reshape(TP, 8, H), (1, 0, 2)).reshape(T_EDGE, H)
    w = (jnp.dot(h_s, w2_ref[...], preferred_element_type=jnp.float32)
         .astype(jnp.bfloat16) + b2_ref[...])
    xp = x_ref[...].astype(jnp.bfloat16)
    x16 = jnp.concatenate(
        [xp[:, MUL * g:MUL * (g + 1)] for g in range(8)], axis=0)
    xrep = jnp.concatenate([x16] * MUL, axis=1)
    tp = jnp.dot(xrep * w, s_ref[...], preferred_element_type=jnp.float32)
    packed = jnp.concatenate(
        [tp[TP * g:TP * (g + 1), :] for g in range(8)], axis=1)
    shp = sh_ref[0]
    shb = jnp.concatenate(
        [jnp.dot(shp, m_ref[:, 128 * u:128 * (u + 1)],
                 preferred_element_type=jnp.float32).reshape(TS, 1, 128)
         for u in range(16)], axis=1).reshape(TP, 128)
    out_ref[...] = packed * (shb * (1.0 / np.sqrt(MUL)))


def _make_tc_edge(off, ntiles):
    return pl.pallas_call(
        _tc_edge_body,
        grid=(ntiles,),
        compiler_params=pltpu.CompilerParams(
            fuse_transposed_lhs_in_matmul=True),
        in_specs=[
            pl.BlockSpec((H, T_EDGE), lambda i: (0, i + off)),
            pl.BlockSpec((TP, 128), lambda i: (i, 0)),
            pl.BlockSpec((1, TS, 128), lambda i: (i + off, 0, 0)),
            pl.BlockSpec((H, H), lambda i: (0, 0)),
            pl.BlockSpec((1, H), lambda i: (0, 0)),
            pl.BlockSpec((H, WN), lambda i: (0, 0)),
            pl.BlockSpec((1, WN), lambda i: (0, 0)),
            pl.BlockSpec((WN, MUL), lambda i: (0, 0)),
            pl.BlockSpec((128, 16 * 128), lambda i: (0, 0)),
        ],
        out_specs=pl.BlockSpec((TP, 128), lambda i: (i, 0)),
        out_shape=jax.ShapeDtypeStruct((ntiles * TP, 128), jnp.float32),
    )


_tc_edge_a = _make_tc_edge(0, TILES_A)
_tc_edge_b = _make_tc_edge(TILES_A, TILES_B)

_S_np = np.zeros((MUL * MUL, MUL), np.float32)
for _i in range(MUL):
    for _j in range(MUL):
        _S_np[_j * MUL + _i, _j] = 1.0
_PERM_np = np.array([(_ij % MUL) * MUL + _ij // MUL
                     for _ij in range(MUL * MUL)], np.int32)
_M_np = np.zeros((128, 16 * 128), np.float32)
for _u in range(16):
    for _g in range(8):
        for _c in range(MUL):
            _M_np[8 * _u + _g, _u * 128 + 16 * _g + _c] = 1.0
_G_np = np.zeros((128, MUL), np.float32)
for _g in range(128 // MUL):
    for _c in range(MUL):
        _G_np[_g * MUL + _c, _c] = 1.0

NR = NPAD * MUL // 128


def _tc_fin_body(psa_ref, pca_ref, psb_ref, pcb_ref, atom_ref,
                 bnw_ref, bnb_ref, g_ref, gt_ref, out_ref):
    s = psa_ref[0] + psa_ref[1] + psb_ref[0] + psb_ref[1]
    cnt = pca_ref[0] + pca_ref[1] + pcb_ref[0] + pcb_ref[1]
    pre = s / jnp.maximum(cnt, 1.0) + atom_ref[...]
    colsum = jnp.sum(pre, axis=0, keepdims=True)
    colsq = jnp.sum(pre * pre, axis=0, keepdims=True)
    chs = jnp.dot(colsum, g_ref[...], preferred_element_type=jnp.float32)
    chsq = jnp.dot(colsq, g_ref[...], preferred_element_type=jnp.float32)
    mean = chs / float(N)
    var = chsq / float(N) - mean * mean
    inv = lax.rsqrt(var + 1e-5)
    scale = bnw_ref[...] * inv
    shift = bnb_ref[...] - mean * scale
    out_ref[...] = (pre * jnp.dot(scale, gt_ref[...], preferred_element_type=jnp.float32)
                    + jnp.dot(shift, gt_ref[...], preferred_element_type=jnp.float32))


_tc_fin = pl.pallas_call(
    _tc_fin_body,
    in_specs=[
        pl.BlockSpec((2, NR, 128), lambda: (0, 0, 0)),
        pl.BlockSpec((2, NR, 128), lambda: (0, 0, 0)),
        pl.BlockSpec((2, NR, 128), lambda: (0, 0, 0)),
        pl.BlockSpec((2, NR, 128), lambda: (0, 0, 0)),
        pl.BlockSpec((NR, 128), lambda: (0, 0)),
        pl.BlockSpec((1, MUL), lambda: (0, 0)),
        pl.BlockSpec((1, MUL), lambda: (0, 0)),
        pl.BlockSpec((128, MUL), lambda: (0, 0)),
        pl.BlockSpec((MUL, 128), lambda: (0, 0)),
    ],
    out_specs=pl.BlockSpec((NR, 128), lambda: (0, 0)),
    out_shape=jax.ShapeDtypeStruct((NR, 128), jnp.float32),
)


def kernel(atom_features, edge_features, edge_sh, edge_index, W1, b1, W2, b2,
           bn_weight, bn_bias):
    gather_a, gather_b, scatter_a, scatter_b = _sc_kernels()
    x3a = gather_a(atom_features, edge_index)
    x3b = gather_b(atom_features, edge_index)

    eft = edge_features.T
    shp = edge_sh[:, 0].reshape(E // T_EDGE, TS, 128)
    perm = jnp.asarray(_PERM_np)
    wargs = (W1.astype(jnp.bfloat16), b1.reshape(1, H),
             W2[:, perm].astype(jnp.bfloat16),
             b2[perm].reshape(1, WN).astype(jnp.bfloat16),
             jnp.asarray(_S_np, jnp.bfloat16), jnp.asarray(_M_np))
    tpa = _tc_edge_a(eft, x3a.reshape(NBLK_A * MUL, 128), shp, *wargs)
    tpb = _tc_edge_b(eft, x3b.reshape(NBLK_B * MUL, 128), shp, *wargs)

    psa, pca = scatter_a(tpa.reshape(NBLK_A, BLK, MUL), edge_index)
    psb, pcb = scatter_b(tpb.reshape(NBLK_B, BLK, MUL), edge_index)

    atom_pad = jnp.pad(atom_features, ((0, NPAD - N), (0, 0))).reshape(NR, 128)
    out = _tc_fin(psa.reshape(2, NR, 128), pca.reshape(2, NR, 128),
                  psb.reshape(2, NR, 128), pcb.reshape(2, NR, 128),
                  atom_pad, bn_weight.reshape(1, MUL), bn_bias.reshape(1, MUL),
                  jnp.asarray(_G_np), jnp.asarray(_G_np.T))
    return out.reshape(NPAD, MUL)[:N]
